# baseline (speedup 1.0000x reference)
import functools

import jax
import jax.numpy as jnp
from jax import lax
from jax.experimental import pallas as pl
from jax.experimental.pallas import tpu as pltpu
from jax.experimental.pallas import tpu_sc as plsc

_D = 768
_SEQ = 8192
_BATCH = 4
_TC_BATCH = 3
_SC_BATCH = _BATCH - _TC_BATCH
_SEQ_BLOCK = 2048

_NC = 2
_NS = 16
_NW = _NC * _NS
_SEQ_PER_W = _SEQ // _NW
_CHUNK_ROWS = 32
_N_CHUNKS = _SEQ_PER_W // _CHUNK_ROWS
_VECS_PER_ROW = _D // 16
_N_UNITS = _N_CHUNKS * _SC_BATCH


def _tc_body(we_ref, pe_ref, out_ref):
    out_ref[...] = we_ref[...] + pe_ref[...][None, :, :]


def _tc_add(word_embeddings, pe):
    n_seq = _SEQ // _SEQ_BLOCK
    return pl.pallas_call(
        _tc_body,
        grid=(n_seq, _TC_BATCH),
        in_specs=[
            pl.BlockSpec((1, _SEQ_BLOCK, _D), lambda s, b: (b, s, 0)),
            pl.BlockSpec((_SEQ_BLOCK, _D), lambda s, b: (s, 0)),
        ],
        out_specs=pl.BlockSpec((1, _SEQ_BLOCK, _D), lambda s, b: (b, s, 0)),
        out_shape=jax.ShapeDtypeStruct((_TC_BATCH, _SEQ, _D), jnp.float32),
        compiler_params=pltpu.CompilerParams(
            dimension_semantics=("arbitrary", "arbitrary"),
        ),
    )(word_embeddings, pe)


def _sc_body(we_hbm, pe_hbm, out_hbm,
             pb0, pb1, wb0, wb1, wb2,
             spe0, spe1, swe0, swe1, swe2, so0, so1, so2):
    pbufs, pe_sems = (pb0, pb1), (spe0, spe1)
    wbufs, we_sems = (wb0, wb1, wb2), (swe0, swe1, swe2)
    out_sems = (so0, so1, so2)
    wid = lax.axis_index("s") * _NC + lax.axis_index("c")
    base_row = wid * _SEQ_PER_W

    def row0(c):
        return pl.multiple_of(base_row + c * _CHUNK_ROWS, 8)

    def issue_pe(c):
        return pltpu.async_copy(
            pe_hbm.at[pl.ds(row0(c), _CHUNK_ROWS), :], pbufs[c % 2], pe_sems[c % 2])

    def issue_we(u):
        c, b = u // _SC_BATCH, u % _SC_BATCH
        return pltpu.async_copy(
            we_hbm.at[_TC_BATCH + b, pl.ds(row0(c), _CHUNK_ROWS), :],
            wbufs[u % 3], we_sems[u % 3])

    def issue_out(u):
        c, b = u // _SC_BATCH, u % _SC_BATCH
        return pltpu.async_copy(
            wbufs[u % 3], out_hbm.at[b, pl.ds(row0(c), _CHUNK_ROWS), :],
            out_sems[u % 3])

    pe_cp = [None, None]
    we_cp = [None, None, None]
    out_cp = [None, None, None]
    pe_cp[0] = issue_pe(0)
    we_cp[0] = issue_we(0)
    pe_waited = [False] * _N_CHUNKS

    for u in range(_N_UNITS):
        c, b = u // _SC_BATCH, u % _SC_BATCH
        if b == 0 and c + 1 < _N_CHUNKS:
            pe_cp[(c + 1) % 2] = issue_pe(c + 1)
        if u + 1 < _N_UNITS:
            if u - 2 >= 0:
                out_cp[(u + 1) % 3].wait()
            we_cp[(u + 1) % 3] = issue_we(u + 1)
        if not pe_waited[c]:
            pe_cp[c % 2].wait()
            pe_waited[c] = True
        we_cp[u % 3].wait()
        wbuf, pbuf = wbufs[u % 3], pbufs[c % 2]

        @plsc.parallel_loop(0, _CHUNK_ROWS * _VECS_PER_ROW, 1, unroll=8)
        def _add(j):
            i = j // _VECS_PER_ROW
            v = (j - i * _VECS_PER_ROW) * 16
            wbuf[i, pl.ds(v, 16)] = wbuf[i, pl.ds(v, 16)] + pbuf[i, pl.ds(v, 16)]

        out_cp[u % 3] = issue_out(u)

    for u in range(max(_N_UNITS - 3, 0), _N_UNITS):
        out_cp[u % 3].wait()


_sc_add = functools.partial(
    pl.kernel,
    out_type=jax.ShapeDtypeStruct((_SC_BATCH, _SEQ, _D), jnp.float32),
    mesh=plsc.VectorSubcoreMesh(core_axis_name="c", subcore_axis_name="s"),
    scratch_types=[
        pltpu.VMEM((_CHUNK_ROWS, _D), jnp.float32),
        pltpu.VMEM((_CHUNK_ROWS, _D), jnp.float32),
        pltpu.VMEM((_CHUNK_ROWS, _D), jnp.float32),
        pltpu.VMEM((_CHUNK_ROWS, _D), jnp.float32),
        pltpu.VMEM((_CHUNK_ROWS, _D), jnp.float32),
        pltpu.SemaphoreType.DMA,
        pltpu.SemaphoreType.DMA,
        pltpu.SemaphoreType.DMA,
        pltpu.SemaphoreType.DMA,
        pltpu.SemaphoreType.DMA,
        pltpu.SemaphoreType.DMA,
        pltpu.SemaphoreType.DMA,
        pltpu.SemaphoreType.DMA,
    ],
)(_sc_body)


def kernel(word_embeddings, pe):
    out_sc = _sc_add(word_embeddings, pe)
    out_tc = _tc_add(word_embeddings, pe)
    return jnp.concatenate([out_tc, out_sc], axis=0)


# speedup vs baseline: 1.4951x; 1.4951x over previous
import functools

import jax
import jax.numpy as jnp
from jax import lax
from jax.experimental import pallas as pl
from jax.experimental.pallas import tpu as pltpu
from jax.experimental.pallas import tpu_sc as plsc

_D = 768
_SEQ = 8192
_BATCH = 4
_TC_BATCH = 0
_SC_BATCH = _BATCH - _TC_BATCH
_SEQ_BLOCK = 2048

_NC = 2
_NS = 16
_NW = _NC * _NS
_SEQ_PER_W = _SEQ // _NW
_CHUNK_ROWS = 32
_N_CHUNKS = _SEQ_PER_W // _CHUNK_ROWS
_VECS_PER_ROW = _D // 16
_N_UNITS = _N_CHUNKS * _SC_BATCH


def _tc_body(we_ref, pe_ref, out_ref):
    out_ref[...] = we_ref[...] + pe_ref[...][None, :, :]


def _tc_add(word_embeddings, pe):
    n_seq = _SEQ // _SEQ_BLOCK
    return pl.pallas_call(
        _tc_body,
        grid=(n_seq, _TC_BATCH),
        in_specs=[
            pl.BlockSpec((1, _SEQ_BLOCK, _D), lambda s, b: (b, s, 0)),
            pl.BlockSpec((_SEQ_BLOCK, _D), lambda s, b: (s, 0)),
        ],
        out_specs=pl.BlockSpec((1, _SEQ_BLOCK, _D), lambda s, b: (b, s, 0)),
        out_shape=jax.ShapeDtypeStruct((_TC_BATCH, _SEQ, _D), jnp.float32),
        compiler_params=pltpu.CompilerParams(
            dimension_semantics=("arbitrary", "arbitrary"),
        ),
    )(word_embeddings, pe)


def _sc_body(we_hbm, pe_hbm, out_hbm,
             pb0, pb1, wb0, wb1, wb2,
             spe0, spe1, swe0, swe1, swe2, so0, so1, so2):
    pbufs, pe_sems = (pb0, pb1), (spe0, spe1)
    wbufs, we_sems = (wb0, wb1, wb2), (swe0, swe1, swe2)
    out_sems = (so0, so1, so2)
    wid = lax.axis_index("s") * _NC + lax.axis_index("c")
    base_row = wid * _SEQ_PER_W

    def row0(c):
        return pl.multiple_of(base_row + c * _CHUNK_ROWS, 8)

    def issue_pe(c):
        return pltpu.async_copy(
            pe_hbm.at[pl.ds(row0(c), _CHUNK_ROWS), :], pbufs[c % 2], pe_sems[c % 2])

    def issue_we(u):
        c, b = u // _SC_BATCH, u % _SC_BATCH
        return pltpu.async_copy(
            we_hbm.at[_TC_BATCH + b, pl.ds(row0(c), _CHUNK_ROWS), :],
            wbufs[u % 3], we_sems[u % 3])

    def issue_out(u):
        c, b = u // _SC_BATCH, u % _SC_BATCH
        return pltpu.async_copy(
            wbufs[u % 3], out_hbm.at[b, pl.ds(row0(c), _CHUNK_ROWS), :],
            out_sems[u % 3])

    pe_cp = [None, None]
    we_cp = [None, None, None]
    out_cp = [None, None, None]
    pe_cp[0] = issue_pe(0)
    we_cp[0] = issue_we(0)
    pe_waited = [False] * _N_CHUNKS

    for u in range(_N_UNITS):
        c, b = u // _SC_BATCH, u % _SC_BATCH
        if b == 0 and c + 1 < _N_CHUNKS:
            pe_cp[(c + 1) % 2] = issue_pe(c + 1)
        if u + 1 < _N_UNITS:
            if u - 2 >= 0:
                out_cp[(u + 1) % 3].wait()
            we_cp[(u + 1) % 3] = issue_we(u + 1)
        if not pe_waited[c]:
            pe_cp[c % 2].wait()
            pe_waited[c] = True
        we_cp[u % 3].wait()
        wbuf, pbuf = wbufs[u % 3], pbufs[c % 2]

        @plsc.parallel_loop(0, _CHUNK_ROWS * _VECS_PER_ROW, 1, unroll=8)
        def _add(j):
            i = j // _VECS_PER_ROW
            v = (j - i * _VECS_PER_ROW) * 16
            plsc.addupdate(wbuf.at[i, pl.ds(v, 16)], pbuf[i, pl.ds(v, 16)])

        out_cp[u % 3] = issue_out(u)

    for u in range(max(_N_UNITS - 3, 0), _N_UNITS):
        out_cp[u % 3].wait()


_sc_add = functools.partial(
    pl.kernel,
    out_type=jax.ShapeDtypeStruct((_SC_BATCH, _SEQ, _D), jnp.float32),
    mesh=plsc.VectorSubcoreMesh(core_axis_name="c", subcore_axis_name="s"),
    scratch_types=[
        pltpu.VMEM((_CHUNK_ROWS, _D), jnp.float32),
        pltpu.VMEM((_CHUNK_ROWS, _D), jnp.float32),
        pltpu.VMEM((_CHUNK_ROWS, _D), jnp.float32),
        pltpu.VMEM((_CHUNK_ROWS, _D), jnp.float32),
        pltpu.VMEM((_CHUNK_ROWS, _D), jnp.float32),
        pltpu.SemaphoreType.DMA,
        pltpu.SemaphoreType.DMA,
        pltpu.SemaphoreType.DMA,
        pltpu.SemaphoreType.DMA,
        pltpu.SemaphoreType.DMA,
        pltpu.SemaphoreType.DMA,
        pltpu.SemaphoreType.DMA,
        pltpu.SemaphoreType.DMA,
    ],
)(_sc_body)


def kernel(word_embeddings, pe):
    return _sc_add(word_embeddings, pe)


# speedup vs baseline: 1.5293x; 1.0228x over previous
import functools

import jax
import jax.numpy as jnp
from jax import lax
from jax.experimental import pallas as pl
from jax.experimental.pallas import tpu as pltpu
from jax.experimental.pallas import tpu_sc as plsc

_D = 768
_SEQ = 8192
_BATCH = 4

_NC = 2
_NS = 16
_NW = _NC * _NS
_SEQ_PER_W = _SEQ // _NW
_CHUNK_ROWS = 16
_N_CHUNKS = _SEQ_PER_W // _CHUNK_ROWS
_VECS_PER_ROW = _D // 16
_N_UNITS = _N_CHUNKS * _BATCH
_NBUF = 6


def _sc_body(we_hbm, pe_hbm, out_hbm, *scratch):
    pbufs, wbufs = scratch[0:2], scratch[2:2 + _NBUF]
    pe_sems = scratch[2 + _NBUF:4 + _NBUF]
    we_sems = scratch[4 + _NBUF:4 + 2 * _NBUF]
    out_sems = scratch[4 + 2 * _NBUF:4 + 3 * _NBUF]
    wid = lax.axis_index("s") * _NC + lax.axis_index("c")
    base_row = wid * _SEQ_PER_W

    def row0(c):
        return pl.multiple_of(base_row + c * _CHUNK_ROWS, 8)

    def issue_pe(c):
        return pltpu.async_copy(
            pe_hbm.at[pl.ds(row0(c), _CHUNK_ROWS), :], pbufs[c % 2], pe_sems[c % 2])

    def issue_we(u):
        c, b = u // _BATCH, u % _BATCH
        return pltpu.async_copy(
            we_hbm.at[b, pl.ds(row0(c), _CHUNK_ROWS), :],
            wbufs[u % _NBUF], we_sems[u % _NBUF])

    def issue_out(u):
        c, b = u // _BATCH, u % _BATCH
        return pltpu.async_copy(
            wbufs[u % _NBUF], out_hbm.at[b, pl.ds(row0(c), _CHUNK_ROWS), :],
            out_sems[u % _NBUF])

    pe_cp = [None, None]
    we_cp = [None] * _NBUF
    out_cp = [None] * _NBUF
    ahead = _NBUF // 2
    pe_cp[0] = issue_pe(0)
    for u in range(ahead):
        we_cp[u % _NBUF] = issue_we(u)
    pe_waited = [False] * _N_CHUNKS

    for u in range(_N_UNITS):
        c, b = u // _BATCH, u % _BATCH
        if b == 0 and c + 1 < _N_CHUNKS:
            pe_cp[(c + 1) % 2] = issue_pe(c + 1)
        if u + ahead < _N_UNITS:
            if u - ahead >= 0:
                out_cp[(u + ahead) % _NBUF].wait()
            we_cp[(u + ahead) % _NBUF] = issue_we(u + ahead)
        if not pe_waited[c]:
            pe_cp[c % 2].wait()
            pe_waited[c] = True
        we_cp[u % _NBUF].wait()
        wbuf, pbuf = wbufs[u % _NBUF], pbufs[c % 2]

        @plsc.parallel_loop(0, _CHUNK_ROWS * _VECS_PER_ROW, 1, unroll=8)
        def _add(j):
            i = j // _VECS_PER_ROW
            v = (j - i * _VECS_PER_ROW) * 16
            plsc.addupdate(wbuf.at[i, pl.ds(v, 16)], pbuf[i, pl.ds(v, 16)])

        out_cp[u % _NBUF] = issue_out(u)

    for u in range(max(_N_UNITS - ahead, 0), _N_UNITS):
        out_cp[u % _NBUF].wait()


_sc_add = functools.partial(
    pl.kernel,
    out_type=jax.ShapeDtypeStruct((_BATCH, _SEQ, _D), jnp.float32),
    mesh=plsc.VectorSubcoreMesh(core_axis_name="c", subcore_axis_name="s"),
    scratch_types=(
        [pltpu.VMEM((_CHUNK_ROWS, _D), jnp.float32)] * (2 + _NBUF)
        + [pltpu.SemaphoreType.DMA] * (2 + 2 * _NBUF)
    ),
)(_sc_body)


def kernel(word_embeddings, pe):
    return _sc_add(word_embeddings, pe)
